# baseline (speedup 1.0000x reference)
import functools

import jax
import jax.numpy as jnp
from jax import lax
from jax.experimental import pallas as pl
from jax.experimental.pallas import tpu as pltpu
from jax.experimental.pallas import tpu_sc as plsc

_CODE_BLK = 256


def _argmin_body(n_tok, dim, cb, x_ref, et_ref, x2_ref, e2_ref,
                 idx_ref, loss_ref, rmax_ref, rarg_ref):
    i = pl.program_id(0)
    x_t = x_ref[...]
    et = et_ref[...]
    x2 = x2_ref[...]
    e2 = e2_ref[...]
    mm = jnp.dot(x_t, et, preferred_element_type=jnp.float32)
    dist = -(x2 - 2.0 * mm + e2)
    cmax = jnp.max(dist, axis=1, keepdims=True)
    iota = lax.broadcasted_iota(jnp.int32, dist.shape, 1)
    carg = jnp.min(jnp.where(dist == cmax, iota, jnp.int32(1 << 30)),
                   axis=1, keepdims=True) + i * cb

    @pl.when(i == 0)
    def _():
        rmax_ref[...] = cmax
        rarg_ref[...] = carg

    @pl.when(i > 0)
    def _():
        rmax = rmax_ref[...]
        better = cmax > rmax
        rarg_ref[...] = jnp.where(better, carg, rarg_ref[...])
        rmax_ref[...] = jnp.where(better, cmax, rmax)

    @pl.when(i == pl.num_programs(0) - 1)
    def _():
        idx_ref[...] = rarg_ref[...]
        loss_ref[0, 0] = -jnp.sum(rmax_ref[...]) * (1.0 / (n_tok * dim))


def _argmin_loss(x, embed):
    n_tok, dim = x.shape
    n_codes = embed.shape[0]
    cb = _CODE_BLK
    nb = n_codes // cb
    et = embed.T
    x2 = jnp.sum(x * x, axis=1, keepdims=True)
    e2 = jnp.sum(embed * embed, axis=1)[None, :]
    idx2, loss = pl.pallas_call(
        functools.partial(_argmin_body, n_tok, dim, cb),
        grid=(nb,),
        in_specs=[
            pl.BlockSpec((n_tok, dim), lambda i: (0, 0)),
            pl.BlockSpec((dim, cb), lambda i: (0, i)),
            pl.BlockSpec((n_tok, 1), lambda i: (0, 0)),
            pl.BlockSpec((1, cb), lambda i: (0, i)),
        ],
        out_specs=[
            pl.BlockSpec((n_tok, 1), lambda i: (0, 0)),
            pl.BlockSpec((1, 1), lambda i: (0, 0), memory_space=pltpu.SMEM),
        ],
        out_shape=[
            jax.ShapeDtypeStruct((n_tok, 1), jnp.int32),
            jax.ShapeDtypeStruct((1, 1), jnp.float32),
        ],
        scratch_shapes=[
            pltpu.VMEM((n_tok, 1), jnp.float32),
            pltpu.VMEM((n_tok, 1), jnp.int32),
        ],
    )(x, et, x2, e2)
    return idx2.reshape(-1), loss.reshape(())


_GATHER_W = 128


@functools.cache
def _make_sc_gather(n_codes, n_tok):
    info = plsc.get_sparse_core_info()
    nw = info.num_cores * info.num_subcores
    b_per_w = n_tok // nw
    mesh = plsc.VectorSubcoreMesh(core_axis_name="c", subcore_axis_name="s")

    @functools.partial(
        pl.kernel,
        mesh=mesh,
        out_type=jax.ShapeDtypeStruct((n_tok, _GATHER_W), jnp.float32),
        scratch_types=[
            pltpu.VMEM((b_per_w,), jnp.int32),
            pltpu.VMEM((b_per_w, _GATHER_W), jnp.float32),
            pltpu.SemaphoreType.DMA,
        ],
    )
    def gather(table_hbm, idx_hbm, out_hbm, idx_v, rows_v, sem):
        wid = lax.axis_index("s") * info.num_cores + lax.axis_index("c")
        base = wid * b_per_w
        pltpu.sync_copy(idx_hbm.at[pl.ds(base, b_per_w)], idx_v)
        pltpu.async_copy(table_hbm.at[idx_v], rows_v, sem).wait()
        pltpu.sync_copy(rows_v, out_hbm.at[pl.ds(base, b_per_w)])

    return gather


def kernel(x, embed):
    n_codes, dim = embed.shape
    indices, commit_loss = _argmin_loss(x, embed)
    table = jnp.pad(embed, ((0, 0), (0, _GATHER_W - dim)))
    quantized = _make_sc_gather(n_codes, x.shape[0])(table, indices)[:, :dim]
    quantized_ste = x + (quantized - x)
    return (quantized_ste, indices, commit_loss)
